# Initial kernel scaffold; baseline (speedup 1.0000x reference)
import jax
import jax.numpy as jnp
from jax.experimental import pallas as pl


def kernel(input_ids, emb_table, pos_table, W_cls, b_cls):
    raise NotImplementedError("write your pallas kernel here")


# speedup vs baseline: 1.8178x; 1.8178x over previous
import functools

import jax
import jax.numpy as jnp
from jax import lax
from jax.experimental import pallas as pl
from jax.experimental.pallas import tpu as pltpu
from jax.experimental.pallas import tpu_sc as plsc

B = 1024
S = 200
VOCAB = 100000
EMB = 128
NUM_LABELS = 42
LP = 48

NC = 2
NS = 16
NW = NC * NS
NTOK = B * S
TW = NTOK // NW
CH = 400
NCHUNK = TW // CH


def _project_body(emb_ref, pos_ref, w_ref, b_ref, e2_ref, p2_ref):
    e2_ref[...] = jnp.dot(emb_ref[...], w_ref[...],
                          preferred_element_type=jnp.float32)

    @pl.when(pl.program_id(0) == 0)
    def _():
        p2 = jnp.dot(pos_ref[...], w_ref[...],
                     preferred_element_type=jnp.float32) + b_ref[...]
        p2_ref[...] = p2[:, :NUM_LABELS]


def _project_tables(emb_table, pos_s, w_pad, b_pad):
    rows_per_blk = 2000
    grid = VOCAB // rows_per_blk
    return pl.pallas_call(
        _project_body,
        grid=(grid,),
        in_specs=[
            pl.BlockSpec((rows_per_blk, EMB), lambda i: (i, 0)),
            pl.BlockSpec((S, EMB), lambda i: (0, 0)),
            pl.BlockSpec((EMB, LP), lambda i: (0, 0)),
            pl.BlockSpec((1, LP), lambda i: (0, 0)),
        ],
        out_specs=[
            pl.BlockSpec((rows_per_blk, LP), lambda i: (i, 0)),
            pl.BlockSpec((S, NUM_LABELS), lambda i: (0, 0)),
        ],
        out_shape=[
            jax.ShapeDtypeStruct((VOCAB, LP), jnp.float32),
            jax.ShapeDtypeStruct((S, NUM_LABELS), jnp.float32),
        ],
    )(emb_table, pos_s, w_pad, b_pad)


def _sc_body(e2_hbm, p2_hbm, ids_hbm, out_hbm, idx_v, pos_v, rows_v, out_v, sem):
    wid = lax.axis_index("s") * NC + lax.axis_index("c")
    base = wid * TW
    pltpu.sync_copy(ids_hbm.at[pl.ds(base, TW)], idx_v)
    pltpu.sync_copy(p2_hbm, pos_v)

    def chunk(c, carry):
        pltpu.async_copy(e2_hbm.at[idx_v.at[pl.ds(c * CH, CH)]], rows_v,
                         sem).wait()
        for h in range(CH // S):
            def row(i, rcarry):
                r = h * S + i
                for off in (0, 16, 26):
                    out_v[pl.ds(r * NUM_LABELS + off, 16)] = (
                        rows_v[r, pl.ds(off, 16)] + pos_v[i, pl.ds(off, 16)])
                return rcarry
            lax.fori_loop(0, S, row, 0)
        pltpu.sync_copy(out_v,
                        out_hbm.at[pl.ds((base + c * CH) * NUM_LABELS,
                                         CH * NUM_LABELS)])
        return carry

    lax.fori_loop(0, NCHUNK, chunk, 0)


@functools.cache
def _sc_gather():
    return pl.kernel(
        _sc_body,
        out_type=jax.ShapeDtypeStruct((NTOK * NUM_LABELS,), jnp.float32),
        mesh=plsc.VectorSubcoreMesh(core_axis_name="c", subcore_axis_name="s",
                                    num_cores=NC, num_subcores=NS),
        scratch_types=[
            pltpu.VMEM((TW,), jnp.int32),
            pltpu.VMEM((S, NUM_LABELS), jnp.float32),
            pltpu.VMEM((CH, LP), jnp.float32),
            pltpu.VMEM((CH * NUM_LABELS,), jnp.float32),
            pltpu.SemaphoreType.DMA,
        ],
        compiler_params=pltpu.CompilerParams(use_tc_tiling_on_sc=False),
    )


def kernel(input_ids, emb_table, pos_table, W_cls, b_cls):
    w_pad = jnp.zeros((EMB, LP), jnp.float32).at[:, :NUM_LABELS].set(W_cls)
    b_pad = jnp.zeros((1, LP), jnp.float32).at[0, :NUM_LABELS].set(b_cls)
    e2, p2 = _project_tables(emb_table, pos_table[:S], w_pad, b_pad)
    ids_flat = input_ids.reshape(-1).astype(jnp.int32)
    out_flat = _sc_gather()(e2, p2, ids_flat)
    return out_flat.reshape(B, S, NUM_LABELS)


# speedup vs baseline: 2.9891x; 1.6443x over previous
import functools

import jax
import jax.numpy as jnp
from jax import lax
from jax.experimental import pallas as pl
from jax.experimental.pallas import tpu as pltpu
from jax.experimental.pallas import tpu_sc as plsc

B = 1024
S = 200
VOCAB = 100000
EMB = 128
NUM_LABELS = 42
LP = 128

NC = 2
NS = 16
NW = NC * NS
NTOK = B * S
SENT_W = B // NW


def _project_body(emb_ref, pos_ref, w_ref, b_ref, e2_ref, p2_ref):
    e2_ref[...] = jnp.dot(emb_ref[...], w_ref[...],
                          preferred_element_type=jnp.float32)

    @pl.when(pl.program_id(0) == 0)
    def _():
        p2_ref[...] = jnp.dot(pos_ref[...], w_ref[...],
                              preferred_element_type=jnp.float32) + b_ref[...]


def _project_tables(emb_table, pos_s, w_pad, b_pad):
    rows_per_blk = 2000
    grid = VOCAB // rows_per_blk
    return pl.pallas_call(
        _project_body,
        grid=(grid,),
        in_specs=[
            pl.BlockSpec((rows_per_blk, EMB), lambda i: (i, 0)),
            pl.BlockSpec((S, EMB), lambda i: (0, 0)),
            pl.BlockSpec((EMB, LP), lambda i: (0, 0)),
            pl.BlockSpec((1, LP), lambda i: (0, 0)),
        ],
        out_specs=[
            pl.BlockSpec((rows_per_blk, LP), lambda i: (i, 0)),
            pl.BlockSpec((S, LP), lambda i: (0, 0)),
        ],
        out_shape=[
            jax.ShapeDtypeStruct((VOCAB, LP), jnp.float32),
            jax.ShapeDtypeStruct((S, LP), jnp.float32),
        ],
    )(emb_table, pos_s, w_pad, b_pad)


def _sc_body(e2_hbm, p2_hbm, ids_hbm, out_hbm, idx_v, pos_v, rows_v, out_v,
             sem):
    wid = lax.axis_index("s") * NC + lax.axis_index("c")
    pltpu.sync_copy(ids_hbm.at[pl.ds(wid * SENT_W * S, SENT_W * S)], idx_v)
    pltpu.sync_copy(p2_hbm, pos_v)

    def sent(j, carry):
        pltpu.async_copy(e2_hbm.at[idx_v.at[pl.ds(j * S, S)]], rows_v,
                         sem).wait()

        def row(r, rcarry):
            for off in (0, 16, 26):
                out_v[r, pl.ds(off, 16)] = (
                    rows_v[r, pl.ds(off, 16)] + pos_v[r, pl.ds(off, 16)])
            return rcarry

        lax.fori_loop(0, S, row, 0)
        pltpu.sync_copy(out_v, out_hbm.at[wid * SENT_W + j])
        return carry

    lax.fori_loop(0, SENT_W, sent, 0)


@functools.cache
def _sc_gather():
    return pl.kernel(
        _sc_body,
        out_type=jax.ShapeDtypeStruct((B, S, NUM_LABELS), jnp.float32),
        mesh=plsc.VectorSubcoreMesh(core_axis_name="c", subcore_axis_name="s",
                                    num_cores=NC, num_subcores=NS),
        scratch_types=[
            pltpu.VMEM((SENT_W * S,), jnp.int32),
            pltpu.VMEM((S, LP), jnp.float32),
            pltpu.VMEM((S, LP), jnp.float32),
            pltpu.VMEM((S, NUM_LABELS), jnp.float32),
            pltpu.SemaphoreType.DMA,
        ],
    )


def kernel(input_ids, emb_table, pos_table, W_cls, b_cls):
    w_pad = jnp.zeros((EMB, LP), jnp.float32).at[:, :NUM_LABELS].set(W_cls)
    b_pad = jnp.zeros((1, LP), jnp.float32).at[0, :NUM_LABELS].set(b_cls)
    e2, p2 = _project_tables(emb_table, pos_table[:S], w_pad, b_pad)
    ids_flat = input_ids.reshape(-1).astype(jnp.int32)
    return _sc_gather()(e2, p2, ids_flat)


# speedup vs baseline: 3.1890x; 1.0669x over previous
import functools

import jax
import jax.numpy as jnp
from jax import lax
from jax.experimental import pallas as pl
from jax.experimental.pallas import tpu as pltpu
from jax.experimental.pallas import tpu_sc as plsc

B = 1024
S = 200
VOCAB = 100000
EMB = 128
NUM_LABELS = 42
LP = 128

NC = 2
NS = 16
NW = NC * NS
NTOK = B * S
SENT_W = B // NW


def _project_body(emb_ref, pos_ref, w_ref, b_ref, e2_ref, p2_ref):
    e2_ref[...] = jnp.dot(emb_ref[...], w_ref[...],
                          preferred_element_type=jnp.float32)

    @pl.when(pl.program_id(0) == 0)
    def _():
        p2_ref[...] = jnp.dot(pos_ref[...], w_ref[...],
                              preferred_element_type=jnp.float32) + b_ref[...]


def _project_tables(emb_table, pos_s, w_pad, b_pad):
    rows_per_blk = 2000
    grid = VOCAB // rows_per_blk
    return pl.pallas_call(
        _project_body,
        grid=(grid,),
        in_specs=[
            pl.BlockSpec((rows_per_blk, EMB), lambda i: (i, 0)),
            pl.BlockSpec((S, EMB), lambda i: (0, 0)),
            pl.BlockSpec((EMB, LP), lambda i: (0, 0)),
            pl.BlockSpec((1, LP), lambda i: (0, 0)),
        ],
        out_specs=[
            pl.BlockSpec((rows_per_blk, LP), lambda i: (i, 0)),
            pl.BlockSpec((S, LP), lambda i: (0, 0)),
        ],
        out_shape=[
            jax.ShapeDtypeStruct((VOCAB, LP), jnp.float32),
            jax.ShapeDtypeStruct((S, LP), jnp.float32),
        ],
    )(emb_table, pos_s, w_pad, b_pad)


def _sc_body(e2_hbm, p2_hbm, ids_hbm, out_hbm, idx_v, pos_v, rows_v, out_v,
             sg0, sg1, so0, so1):
    wid = lax.axis_index("s") * NC + lax.axis_index("c")
    pltpu.sync_copy(ids_hbm.at[pl.ds(wid * SENT_W * S, SENT_W * S)], idx_v)
    pltpu.sync_copy(p2_hbm, pos_v)
    sg = (sg0, sg1)
    so = (so0, so1)

    def start_gather(j, b):
        pltpu.async_copy(e2_hbm.at[idx_v.at[pl.ds(j * S, S)]],
                         rows_v.at[b], sg[b])

    start_gather(0, 0)
    start_gather(1, 1)

    def pair(g, carry):
        for b in (0, 1):
            j = 2 * g + b
            pltpu.make_async_copy(e2_hbm.at[idx_v.at[pl.ds(j * S, S)]],
                                  rows_v.at[b], sg[b]).wait()

            @pl.when(g > 0)
            def _():
                pltpu.make_async_copy(out_v.at[b],
                                      out_hbm.at[wid * SENT_W + j],
                                      so[b]).wait()

            def row(r, rcarry):
                for off in (0, 16, 26):
                    out_v[b, r, pl.ds(off, 16)] = (
                        rows_v[b, r, pl.ds(off, 16)]
                        + pos_v[pl.ds(r * 48 + off, 16)])
                return rcarry

            lax.fori_loop(0, S, row, 0, unroll=8)
            pltpu.async_copy(out_v.at[b], out_hbm.at[wid * SENT_W + j],
                             so[b])

            @pl.when(g < SENT_W // 2 - 1)
            def _():
                start_gather(j + 2, b)
        return carry

    lax.fori_loop(0, SENT_W // 2, pair, 0)
    for b in (0, 1):
        pltpu.make_async_copy(out_v.at[b],
                              out_hbm.at[wid * SENT_W + SENT_W - 2 + b],
                              so[b]).wait()


@functools.cache
def _sc_gather():
    return pl.kernel(
        _sc_body,
        out_type=jax.ShapeDtypeStruct((B, S, NUM_LABELS), jnp.float32),
        mesh=plsc.VectorSubcoreMesh(core_axis_name="c", subcore_axis_name="s",
                                    num_cores=NC, num_subcores=NS),
        scratch_types=[
            pltpu.VMEM((SENT_W * S,), jnp.int32),
            pltpu.VMEM((S * 48,), jnp.float32),
            pltpu.VMEM((2, S, LP), jnp.float32),
            pltpu.VMEM((2, S, NUM_LABELS), jnp.float32),
            pltpu.SemaphoreType.DMA,
            pltpu.SemaphoreType.DMA,
            pltpu.SemaphoreType.DMA,
            pltpu.SemaphoreType.DMA,
        ],
    )


def kernel(input_ids, emb_table, pos_table, W_cls, b_cls):
    w_pad = jnp.zeros((EMB, LP), jnp.float32).at[:, :NUM_LABELS].set(W_cls)
    b_pad = jnp.zeros((1, LP), jnp.float32).at[0, :NUM_LABELS].set(b_cls)
    e2, p2 = _project_tables(emb_table, pos_table[:S], w_pad, b_pad)
    p2_flat = p2[:, :48].reshape(-1)
    ids_flat = input_ids.reshape(-1).astype(jnp.int32)
    return _sc_gather()(e2, p2_flat, ids_flat)
